# baseline (speedup 1.0000x reference)
import functools

import jax
import jax.numpy as jnp
from jax import lax
from jax.experimental import pallas as pl
from jax.experimental.pallas import tpu as pltpu
from jax.experimental.pallas import tpu_sc as plsc

B = 16384
N_FEAT = 39
N_NUM = 13
N_CAT = 26
CARD = 100000
D = 16

NC = 2
NS = 16
NW = NC * NS
BPW = B // NW
GROUPS = BPW // 16
STREAM = 128
NSTREAM = BPW // STREAM

MAGIC = 1.5 * 2.0**23


def _body(xT_hbm, w_hbm, b_hbm, tab_hbm, out_hbm,
          xv, idxv, wv, bv, catv, numv, gsem, wsem):
    wid = lax.axis_index("s") * NC + lax.axis_index("c")
    base = wid * BPW

    pltpu.sync_copy(xT_hbm.at[:, pl.ds(base, BPW)], xv)
    pltpu.sync_copy(w_hbm, wv)
    pltpu.sync_copy(b_hbm, bv)

    def idx_body(i, _):
        f = i >> 5
        g = i & (GROUPS - 1)
        v = xv[N_NUM + f, pl.ds(g * 16, 16)]
        v = jnp.where(v != v, 0.0, v)
        r = (v + MAGIC) - MAGIC
        r = jnp.minimum(r, float(CARD - 1))
        r = jnp.maximum(r, 0.0)
        idxv[pl.ds(i * 16, 16)] = r.astype(jnp.int32) + f * CARD
        return 0
    lax.fori_loop(0, N_CAT * GROUPS, idx_body, 0)

    for f in range(N_CAT):
        cps = [
            pltpu.async_copy(
                tab_hbm.at[idxv.at[pl.ds(f * BPW + j * STREAM, STREAM)]],
                catv.at[pl.ds(j * STREAM, STREAM)],
                gsem)
            for j in range(NSTREAM)
        ]
        for cp in cps:
            cp.wait()
        pltpu.sync_copy(catv, out_hbm.at[pl.ds(base, BPW), pl.ds(N_NUM + f, 1)])

    for f in range(N_NUM):
        wf = wv[f, :]
        bf = bv[f, :]

        def num_body(g, _):
            xg = xv[f, pl.ds(g * 16, 16)]
            for l in range(16):
                numv[g * 16 + l, 0, :] = xg[l] * wf + bf
            return 0
        lax.fori_loop(0, GROUPS, num_body, 0)
        pltpu.sync_copy(numv, out_hbm.at[pl.ds(base, BPW), pl.ds(f, 1)])


@functools.partial(
    pl.kernel,
    out_type=jax.ShapeDtypeStruct((B, N_FEAT, D), jnp.float32),
    mesh=plsc.VectorSubcoreMesh(core_axis_name="c", subcore_axis_name="s"),
    compiler_params=pltpu.CompilerParams(use_tc_tiling_on_sc=False),
    scratch_types=[
        pltpu.VMEM((N_FEAT, BPW), jnp.float32),
        pltpu.VMEM((N_CAT * BPW,), jnp.int32),
        pltpu.VMEM((N_NUM, D), jnp.float32),
        pltpu.VMEM((N_NUM, D), jnp.float32),
        pltpu.VMEM((BPW, 1, D), jnp.float32),
        pltpu.VMEM((BPW, 1, D), jnp.float32),
        pltpu.SemaphoreType.DMA,
        pltpu.SemaphoreType.DMA,
    ],
)
def _sc_embed(xT_hbm, w_hbm, b_hbm, tab_hbm, out_hbm,
              xv, idxv, wv, bv, catv, numv, gsem, wsem):
    _body(xT_hbm, w_hbm, b_hbm, tab_hbm, out_hbm,
          xv, idxv, wv, bv, catv, numv, gsem, wsem)


def kernel(x, num_W, num_b, cat_tables):
    xT = x.T
    tab = cat_tables.reshape(N_CAT * CARD, 1, D)
    return _sc_embed(xT, num_W, num_b, tab)


# speedup vs baseline: 6.1866x; 6.1866x over previous
import functools

import jax
import jax.numpy as jnp
from jax import lax
from jax.experimental import pallas as pl
from jax.experimental.pallas import tpu as pltpu
from jax.experimental.pallas import tpu_sc as plsc

B = 16384
N_FEAT = 39
N_NUM = 13
N_CAT = 26
CARD = 100000
D = 16

NC = 2
NS = 16
NW = NC * NS
BPW = B // NW
GROUPS = BPW // 16
STREAM = 128
NSTR = BPW // STREAM
GPS = STREAM // 16

MAGIC = 1.5 * 2.0**23


def _body(x_hbm, w_hbm, b_hbm, tab_hbm, out_hbm,
          xv, gidx, oidx, wv, bv, catv, numv, gsem, ssem0, ssem1,
          nsem0, nsem1):
    ssem = (ssem0, ssem1)
    nsem = (nsem0, nsem1)
    wid = lax.axis_index("s") * NC + lax.axis_index("c")
    base = wid * BPW

    pltpu.sync_copy(x_hbm.at[pl.ds(base, BPW)], xv)
    pltpu.sync_copy(w_hbm, wv)
    pltpu.sync_copy(b_hbm, bv)

    iota = lax.iota(jnp.int32, 16)
    iota39 = iota * N_FEAT

    def cat_idx_body(i, _):
        f = i >> 5
        g = i & (GROUPS - 1)
        rows = iota + g * 16
        v = plsc.load_gather(xv, [rows, jnp.full((16,), N_NUM + f, jnp.int32)])
        v = jnp.where(v != v, 0.0, v)
        r = (v + MAGIC) - MAGIC
        r = jnp.minimum(r, float(CARD - 1))
        r = jnp.maximum(r, 0.0)
        row = f * NSTR + (g >> 3)
        col = (g & (GPS - 1)) * 16
        gidx[row, pl.ds(col, 16)] = r.astype(jnp.int32) + f * CARD
        oidx[row, pl.ds(col, 16)] = iota39 + ((base + g * 16) * N_FEAT
                                              + N_NUM + f)
        return 0
    lax.fori_loop(0, N_CAT * GROUPS, cat_idx_body, 0)

    def num_idx_body(i, _):
        f = i >> 5
        g = i & (GROUPS - 1)
        row = N_CAT * NSTR + f * NSTR + (g >> 3)
        col = (g & (GPS - 1)) * 16
        oidx[row, pl.ds(col, 16)] = iota39 + ((base + g * 16) * N_FEAT + f)
        return 0
    lax.fori_loop(0, N_NUM * GROUPS, num_idx_body, 0)

    def fire_gathers(f):
        return [
            pltpu.async_copy(tab_hbm.at[gidx.at[f * NSTR + j]],
                             catv.at[f % 2, pl.ds(j * STREAM, STREAM)],
                             gsem)
            for j in range(NSTR)
        ]

    def fire_scatters(f):
        return [
            pltpu.async_copy(catv.at[f % 2, pl.ds(j * STREAM, STREAM)],
                             out_hbm.at[oidx.at[f * NSTR + j]],
                             ssem[f % 2])
            for j in range(NSTR)
        ]

    pend_g = fire_gathers(0)
    pend_s = [None, None]
    for f in range(N_CAT):
        for cp in pend_g:
            cp.wait()
        new_s = fire_scatters(f)
        if f + 1 < N_CAT:
            if pend_s[(f + 1) % 2] is not None:
                for cp in pend_s[(f + 1) % 2]:
                    cp.wait()
            pend_g = fire_gathers(f + 1)
        pend_s[f % 2] = new_s

    num_pend = [None, None]
    for f in range(N_NUM):
        wf = wv[f, :]
        bf = bv[f, :]
        if num_pend[f % 2] is not None:
            for cp in num_pend[f % 2]:
                cp.wait()

        def num_body(g, _):
            rows = iota + g * 16
            xg = plsc.load_gather(xv, [rows, jnp.full((16,), f, jnp.int32)])
            for l in range(16):
                numv[f % 2, g * 16 + l, :] = xg[l] * wf + bf
            return 0
        lax.fori_loop(0, GROUPS, num_body, 0)
        num_pend[f % 2] = [
            pltpu.async_copy(numv.at[f % 2, pl.ds(j * STREAM, STREAM)],
                             out_hbm.at[oidx.at[(N_CAT + f) * NSTR + j]],
                             nsem[f % 2])
            for j in range(NSTR)
        ]

    for pend in (num_pend[0], num_pend[1], pend_s[0], pend_s[1]):
        if pend is not None:
            for cp in pend:
                cp.wait()


@functools.partial(
    pl.kernel,
    out_type=jax.ShapeDtypeStruct((B * N_FEAT, D), jnp.float32),
    mesh=plsc.VectorSubcoreMesh(core_axis_name="c", subcore_axis_name="s"),
    compiler_params=pltpu.CompilerParams(use_tc_tiling_on_sc=False, needs_layout_passes=False),
    scratch_types=[
        pltpu.VMEM((BPW, N_FEAT), jnp.float32),
        pltpu.VMEM((N_CAT * NSTR, STREAM), jnp.int32),
        pltpu.VMEM((N_FEAT * NSTR, STREAM), jnp.int32),
        pltpu.VMEM((N_NUM, D), jnp.float32),
        pltpu.VMEM((N_NUM, D), jnp.float32),
        pltpu.VMEM((2, BPW, D), jnp.float32),
        pltpu.VMEM((2, BPW, D), jnp.float32),
        pltpu.SemaphoreType.DMA,
        pltpu.SemaphoreType.DMA,
        pltpu.SemaphoreType.DMA,
        pltpu.SemaphoreType.DMA,
        pltpu.SemaphoreType.DMA,
    ],
)
def _sc_embed(x_hbm, w_hbm, b_hbm, tab_hbm, out_hbm,
              xv, gidx, oidx, wv, bv, catv, numv,
              gsem, ssem0, ssem1, nsem0, nsem1):
    _body(x_hbm, w_hbm, b_hbm, tab_hbm, out_hbm,
          xv, gidx, oidx, wv, bv, catv, numv,
          gsem, ssem0, ssem1, nsem0, nsem1)


def kernel(x, num_W, num_b, cat_tables):
    tab = cat_tables.reshape(N_CAT * CARD, D)
    out = _sc_embed(x, num_W, num_b, tab)
    return out.reshape(B, N_FEAT, D)


# speedup vs baseline: 8.4441x; 1.3649x over previous
import functools

import jax
import jax.numpy as jnp
from jax import lax
from jax.experimental import pallas as pl
from jax.experimental.pallas import tpu as pltpu
from jax.experimental.pallas import tpu_sc as plsc

B = 16384
N_FEAT = 39
N_NUM = 13
N_CAT = 26
CARD = 100000
D = 16

NC = 2
NS = 16
NW = NC * NS
BPW = B // NW
CHUNK = 64
NCHUNK = BPW // CHUNK
GPC = CHUNK // 16

MAGIC = 1.5 * 2.0**23


def _body(x_hbm, w_hbm, b_hbm, tab_hbm, out_hbm,
          xc, gidx, wv, bv, catv, buf, gsem, osem0, osem1):
    osem = (osem0, osem1)
    wid = lax.axis_index("s") * NC + lax.axis_index("c")
    base = wid * BPW

    pltpu.sync_copy(w_hbm, wv)
    pltpu.sync_copy(b_hbm, bv)

    iota = lax.iota(jnp.int32, 16)

    pend_o = [None, None]
    for c in range(NCHUNK):
        sel = c % 2
        row0 = base + c * CHUNK
        pltpu.sync_copy(x_hbm.at[pl.ds(row0, CHUNK)], xc)

        def cat_idx_body(i, _):
            f = i >> 2
            g = i & (GPC - 1)
            rows = iota + g * 16
            v = plsc.load_gather(
                xc, [rows, jnp.full((16,), N_NUM + f, jnp.int32)])
            v = jnp.where(v != v, 0.0, v)
            r = (v + MAGIC) - MAGIC
            r = jnp.minimum(r, float(CARD - 1))
            r = jnp.maximum(r, 0.0)
            gidx[f, pl.ds(g * 16, 16)] = r.astype(jnp.int32)
            return 0
        lax.fori_loop(0, N_CAT * GPC, cat_idx_body, 0)

        if pend_o[sel] is not None:
            pend_o[sel].wait()
            pend_o[sel] = None

        gcps = [
            pltpu.async_copy(tab_hbm.at[f].at[gidx.at[f]], catv.at[f], gsem)
            for f in range(N_CAT)
        ]

        def num_body(i, _):
            f = i >> 2
            g = i & (GPC - 1)
            rows = iota + g * 16
            xg = plsc.load_gather(xc, [rows, jnp.full((16,), f, jnp.int32)])
            wf = wv[f, :]
            bf = bv[f, :]
            for l in range(16):
                buf[sel, g * 16 + l, f, :] = xg[l] * wf + bf
            return 0
        lax.fori_loop(0, N_NUM * GPC, num_body, 0)

        for cp in gcps:
            cp.wait()

        def cat_copy_body(i, _):
            for f in range(N_CAT):
                buf[sel, i, N_NUM + f, :] = catv[f, i, :]
            return 0
        lax.fori_loop(0, CHUNK, cat_copy_body, 0)

        pend_o[sel] = pltpu.async_copy(
            buf.at[sel], out_hbm.at[pl.ds(row0, CHUNK)], osem[sel])

    for pend in pend_o:
        if pend is not None:
            pend.wait()


@functools.partial(
    pl.kernel,
    out_type=jax.ShapeDtypeStruct((B, N_FEAT, D), jnp.float32),
    mesh=plsc.VectorSubcoreMesh(core_axis_name="c", subcore_axis_name="s"),
    compiler_params=pltpu.CompilerParams(
        use_tc_tiling_on_sc=False, needs_layout_passes=False),
    scratch_types=[
        pltpu.VMEM((CHUNK, N_FEAT), jnp.float32),
        pltpu.VMEM((N_CAT, CHUNK), jnp.int32),
        pltpu.VMEM((N_NUM, D), jnp.float32),
        pltpu.VMEM((N_NUM, D), jnp.float32),
        pltpu.VMEM((N_CAT, CHUNK, D), jnp.float32),
        pltpu.VMEM((2, CHUNK, N_FEAT, D), jnp.float32),
        pltpu.SemaphoreType.DMA,
        pltpu.SemaphoreType.DMA,
        pltpu.SemaphoreType.DMA,
    ],
)
def _sc_embed(x_hbm, w_hbm, b_hbm, tab_hbm, out_hbm,
              xc, gidx, wv, bv, catv, buf, gsem, osem0, osem1):
    _body(x_hbm, w_hbm, b_hbm, tab_hbm, out_hbm,
          xc, gidx, wv, bv, catv, buf, gsem, osem0, osem1)


def kernel(x, num_W, num_b, cat_tables):
    return _sc_embed(x, num_W, num_b, cat_tables)


# speedup vs baseline: 33.0421x; 3.9130x over previous
import functools

import jax
import jax.numpy as jnp
from jax import lax
from jax.experimental import pallas as pl
from jax.experimental.pallas import tpu as pltpu
from jax.experimental.pallas import tpu_sc as plsc

B = 16384
N_FEAT = 39
N_NUM = 13
N_CAT = 26
CARD = 100000
D = 16

NC = 2
NS = 16
NW = NC * NS
BPW = B // NW
GROUPS = BPW // 16
KCH = 128
NK = BPW // KCH
GPK = KCH // 16
CACHE = 64

MAGIC = 1.5 * 2.0**23


def _body(x_hbm, w_hbm, b_hbm, tab_hbm, out_hbm,
          xv, gidx, wv, bv, cachev, cbuf, nbuf, mflag, gsem):
    wid = lax.axis_index("s") * NC + lax.axis_index("c")
    base = wid * BPW

    pltpu.sync_copy(x_hbm.at[:, pl.ds(base, BPW)], xv)
    pltpu.sync_copy(w_hbm, wv)
    pltpu.sync_copy(b_hbm, bv)
    pltpu.sync_copy(tab_hbm.at[:, :, pl.ds(0, CACHE)], cachev)

    def cat_idx_body(i, _):
        f = i >> 2
        k = i & (NK - 1)
        m = jnp.int32(0)
        for j in range(GPK):
            v = xv[N_NUM + f, pl.ds((k * GPK + j) * 16, 16)]
            v = jnp.where(v != v, 0.0, v)
            r = (v + MAGIC) - MAGIC
            r = jnp.minimum(r, float(CARD - 1))
            r = jnp.maximum(r, 0.0)
            ri = r.astype(jnp.int32)
            gidx[f, k, pl.ds(j * 16, 16)] = ri
            m = jnp.maximum(m, jnp.max(ri))
        mflag[f, k] = m
        return 0
    lax.fori_loop(0, N_CAT * NK, cat_idx_body, 0)

    for f in range(N_NUM):
        wf = wv[f, :]
        bf = bv[f, :]

        def num_body(g, _):
            xg = xv[f, pl.ds(g * 16, 16)]
            for d in range(D):
                nbuf[d, pl.ds(g * 16, 16)] = xg * wf[d] + bf[d]
            return 0
        lax.fori_loop(0, GROUPS, num_body, 0)
        pltpu.sync_copy(nbuf, out_hbm.at[f, :, pl.ds(base, BPW)])

    def cat_body(cf, _):
        cf16 = jnp.full((16,), cf, jnp.int32)

        def grp_body(g, _):
            idxg = gidx[cf, g >> 3, pl.ds((g & 7) * 16, 16)]
            idxc = jnp.minimum(idxg, CACHE - 1)
            for d in range(D):
                val = plsc.load_gather(
                    cachev, [cf16, jnp.full((16,), d, jnp.int32), idxc])
                cbuf[d, pl.ds(g * 16, 16)] = val
            return 0
        lax.fori_loop(0, GROUPS, grp_body, 0)

        for k in range(NK):
            @pl.when(mflag[cf, k] >= CACHE)
            def _():
                for d in range(D):
                    pltpu.async_copy(
                        tab_hbm.at[cf, d].at[gidx.at[cf, k]],
                        cbuf.at[d, pl.ds(k * KCH, KCH)],
                        gsem)
                pltpu.make_async_copy(
                    tab_hbm.at[0, :, pl.ds(0, KCH)],
                    cbuf.at[:, pl.ds(0, KCH)], gsem).wait()

        pltpu.sync_copy(cbuf, out_hbm.at[N_NUM + cf, :, pl.ds(base, BPW)])
        return 0
    lax.fori_loop(0, N_CAT, cat_body, 0)


@functools.partial(
    pl.kernel,
    out_type=jax.ShapeDtypeStruct((N_FEAT, D, B), jnp.float32),
    mesh=plsc.VectorSubcoreMesh(core_axis_name="c", subcore_axis_name="s"),
    compiler_params=pltpu.CompilerParams(
        use_tc_tiling_on_sc=False, needs_layout_passes=False),
    scratch_types=[
        pltpu.VMEM((N_FEAT, BPW), jnp.float32),
        pltpu.VMEM((N_CAT, NK, KCH), jnp.int32),
        pltpu.VMEM((N_NUM, D), jnp.float32),
        pltpu.VMEM((N_NUM, D), jnp.float32),
        pltpu.VMEM((N_CAT, D, CACHE), jnp.float32),
        pltpu.VMEM((D, BPW), jnp.float32),
        pltpu.VMEM((D, BPW), jnp.float32),
        pltpu.SMEM((N_CAT, NK), jnp.int32),
        pltpu.SemaphoreType.DMA,
    ],
)
def _sc_embed(x_hbm, w_hbm, b_hbm, tab_hbm, out_hbm,
              xv, gidx, wv, bv, cachev, cbuf, nbuf, mflag, gsem):
    _body(x_hbm, w_hbm, b_hbm, tab_hbm, out_hbm,
          xv, gidx, wv, bv, cachev, cbuf, nbuf, mflag, gsem)


def kernel(x, num_W, num_b, cat_tables):
    out_t = _sc_embed(x.T, num_W, num_b, cat_tables.transpose(0, 2, 1))
    return out_t.transpose(2, 0, 1)


# speedup vs baseline: 34.3186x; 1.0386x over previous
import functools

import jax
import jax.numpy as jnp
from jax import lax
from jax.experimental import pallas as pl
from jax.experimental.pallas import tpu as pltpu
from jax.experimental.pallas import tpu_sc as plsc

B = 16384
N_FEAT = 39
N_NUM = 13
N_CAT = 26
CARD = 100000
D = 16

NC = 2
NS = 16
NW = NC * NS
BPW = B // NW
GROUPS = BPW // 16
KCH = 128
NK = BPW // KCH
GPK = KCH // 16
CACHE = 64

MAGIC = 1.5 * 2.0**23


def _body(x_hbm, w_hbm, b_hbm, tab_hbm, out_hbm,
          xv, gidx, wv, bv, sharedv, cachev, cbuf, nbuf, mflag,
          gsem, cachesem, csem0, csem1, nsem0, nsem1):
    sid = lax.axis_index("s")
    wid = sid * NC + lax.axis_index("c")
    base = wid * BPW
    nsem = (nsem0, nsem1)
    csem = (csem0, csem1)

    @pl.when(sid == 0)
    def _():
        pltpu.async_copy(tab_hbm.at[:, :, pl.ds(0, CACHE)], sharedv, cachesem)

    pltpu.sync_copy(x_hbm.at[:, pl.ds(base, BPW)], xv)
    pltpu.sync_copy(w_hbm, wv)
    pltpu.sync_copy(b_hbm, bv)

    def cat_idx_body(i, _):
        f = i >> 2
        k = i & (NK - 1)
        m = jnp.int32(0)
        for j in range(GPK):
            v = xv[N_NUM + f, pl.ds((k * GPK + j) * 16, 16)]
            v = jnp.where(v != v, 0.0, v)
            r = (v + MAGIC) - MAGIC
            r = jnp.minimum(r, float(CARD - 1))
            r = jnp.maximum(r, 0.0)
            ri = r.astype(jnp.int32)
            gidx[f, k, pl.ds(j * 16, 16)] = ri
            m = jnp.maximum(m, jnp.max(ri))
        mflag[f, k] = m
        return 0
    lax.fori_loop(0, N_CAT * NK, cat_idx_body, 0)

    for f in range(N_NUM):
        sel = f & 1
        wf = wv[f, :]
        bf = bv[f, :]
        if f >= 2:
            pltpu.make_async_copy(
                nbuf.at[sel], out_hbm.at[f - 2, :, pl.ds(base, BPW)],
                nsem[sel]).wait()

        def num_body(g, _):
            xg = xv[f, pl.ds(g * 16, 16)]
            for d in range(D):
                nbuf[sel, d, pl.ds(g * 16, 16)] = xg * wf[d] + bf[d]
            return 0
        lax.fori_loop(0, GROUPS, num_body, 0)
        pltpu.async_copy(
            nbuf.at[sel], out_hbm.at[f, :, pl.ds(base, BPW)], nsem[sel])

    @pl.when(sid == 0)
    def _():
        pltpu.make_async_copy(
            tab_hbm.at[:, :, pl.ds(0, CACHE)], sharedv, cachesem).wait()
    plsc.subcore_barrier()
    pltpu.sync_copy(sharedv, cachev)

    def compute_cat(cf, buf):
        cf16 = jnp.full((16,), cf, jnp.int32)

        def grp_body(g, _):
            idxg = gidx[cf, g >> 3, pl.ds((g & 7) * 16, 16)]
            idxc = jnp.minimum(idxg, CACHE - 1)
            for d in range(D):
                val = plsc.load_gather(
                    cachev, [cf16, jnp.full((16,), d, jnp.int32), idxc])
                buf[d, pl.ds(g * 16, 16)] = val
            return 0
        lax.fori_loop(0, GROUPS, grp_body, 0)

        for k in range(NK):
            @pl.when(mflag[cf, k] >= CACHE)
            def _():
                for d in range(D):
                    pltpu.async_copy(
                        tab_hbm.at[cf, d].at[gidx.at[cf, k]],
                        buf.at[d, pl.ds(k * KCH, KCH)],
                        gsem)
                pltpu.make_async_copy(
                    tab_hbm.at[0, :, pl.ds(0, KCH)],
                    buf.at[:, pl.ds(0, KCH)], gsem).wait()

    def cat_pair(j, _):
        cf0 = 2 * j
        cf1 = 2 * j + 1

        @pl.when(j >= 1)
        def _():
            pltpu.make_async_copy(
                cbuf.at[0], out_hbm.at[N_NUM, :, pl.ds(base, BPW)],
                csem[0]).wait()
        compute_cat(cf0, cbuf.at[0])
        pltpu.async_copy(
            cbuf.at[0], out_hbm.at[N_NUM + cf0, :, pl.ds(base, BPW)], csem[0])

        @pl.when(j >= 1)
        def _():
            pltpu.make_async_copy(
                cbuf.at[1], out_hbm.at[N_NUM, :, pl.ds(base, BPW)],
                csem[1]).wait()
        compute_cat(cf1, cbuf.at[1])
        pltpu.async_copy(
            cbuf.at[1], out_hbm.at[N_NUM + cf1, :, pl.ds(base, BPW)], csem[1])
        return 0
    lax.fori_loop(0, N_CAT // 2, cat_pair, 0)

    for sel in range(2):
        pltpu.make_async_copy(
            nbuf.at[sel], out_hbm.at[0, :, pl.ds(base, BPW)],
            nsem[sel]).wait()
        pltpu.make_async_copy(
            cbuf.at[sel], out_hbm.at[N_NUM, :, pl.ds(base, BPW)],
            csem[sel]).wait()


@functools.partial(
    pl.kernel,
    out_type=jax.ShapeDtypeStruct((N_FEAT, D, B), jnp.float32),
    mesh=plsc.VectorSubcoreMesh(core_axis_name="c", subcore_axis_name="s"),
    compiler_params=pltpu.CompilerParams(
        use_tc_tiling_on_sc=False, needs_layout_passes=False),
    scratch_types=[
        pltpu.VMEM((N_FEAT, BPW), jnp.float32),
        pltpu.VMEM((N_CAT, NK, KCH), jnp.int32),
        pltpu.VMEM((N_NUM, D), jnp.float32),
        pltpu.VMEM((N_NUM, D), jnp.float32),
        pltpu.VMEM_SHARED((N_CAT, D, CACHE), jnp.float32),
        pltpu.VMEM((N_CAT, D, CACHE), jnp.float32),
        pltpu.VMEM((2, D, BPW), jnp.float32),
        pltpu.VMEM((2, D, BPW), jnp.float32),
        pltpu.SMEM((N_CAT, NK), jnp.int32),
        pltpu.SemaphoreType.DMA,
        pltpu.SemaphoreType.DMA,
        pltpu.SemaphoreType.DMA,
        pltpu.SemaphoreType.DMA,
        pltpu.SemaphoreType.DMA,
        pltpu.SemaphoreType.DMA,
    ],
)
def _sc_embed(x_hbm, w_hbm, b_hbm, tab_hbm, out_hbm,
              xv, gidx, wv, bv, sharedv, cachev, cbuf, nbuf, mflag,
              gsem, cachesem, csem0, csem1, nsem0, nsem1):
    _body(x_hbm, w_hbm, b_hbm, tab_hbm, out_hbm,
          xv, gidx, wv, bv, sharedv, cachev, cbuf, nbuf, mflag,
          gsem, cachesem, csem0, csem1, nsem0, nsem1)


def kernel(x, num_W, num_b, cat_tables):
    out_t = _sc_embed(x.T, num_W, num_b, cat_tables.transpose(0, 2, 1))
    return out_t.transpose(2, 0, 1)


# speedup vs baseline: 34.3848x; 1.0019x over previous
import functools

import jax
import jax.numpy as jnp
from jax import lax
from jax.experimental import pallas as pl
from jax.experimental.pallas import tpu as pltpu
from jax.experimental.pallas import tpu_sc as plsc

B = 16384
N_FEAT = 39
N_NUM = 13
N_CAT = 26
CARD = 100000
D = 16

NC = 2
NS = 16
NW = NC * NS
BPW = B // NW
GROUPS = BPW // 16
KCH = 128
NK = BPW // KCH
GPK = KCH // 16
CACHE = 64

MAGIC = 1.5 * 2.0**23


def _body(x_hbm, w_hbm, b_hbm, tab_hbm, out_hbm,
          xv, gidx, wv, bv, sharedv, cachev, cbuf, nbuf, mflag,
          gsem, cachesem, csem0, csem1, nsem0, nsem1):
    sid = lax.axis_index("s")
    wid = sid * NC + lax.axis_index("c")
    base = wid * BPW
    nsem = (nsem0, nsem1)
    csem = (csem0, csem1)

    pltpu.async_copy(
        tab_hbm.at[:, pl.ds(sid, 1), pl.ds(0, CACHE)],
        sharedv.at[:, pl.ds(sid, 1), :], cachesem)

    pltpu.sync_copy(x_hbm.at[:, pl.ds(base, BPW)], xv)
    pltpu.sync_copy(w_hbm, wv)
    pltpu.sync_copy(b_hbm, bv)

    def cat_idx_body(i, _):
        f = i >> 2
        k = i & (NK - 1)
        m = jnp.int32(0)
        for j in range(GPK):
            v = xv[N_NUM + f, pl.ds((k * GPK + j) * 16, 16)]
            v = jnp.where(v != v, 0.0, v)
            r = (v + MAGIC) - MAGIC
            r = jnp.minimum(r, float(CARD - 1))
            r = jnp.maximum(r, 0.0)
            ri = r.astype(jnp.int32)
            gidx[f, k, pl.ds(j * 16, 16)] = ri
            m = jnp.maximum(m, jnp.max(ri))
        mflag[f, k] = m
        return 0
    lax.fori_loop(0, N_CAT * NK, cat_idx_body, 0)

    for f in range(N_NUM):
        sel = f & 1
        wf = wv[f, :]
        bf = bv[f, :]
        if f >= 2:
            pltpu.make_async_copy(
                nbuf.at[sel], out_hbm.at[f - 2, :, pl.ds(base, BPW)],
                nsem[sel]).wait()

        def num_body(g, _):
            xg = xv[f, pl.ds(g * 16, 16)]
            for d in range(D):
                nbuf[sel, d, pl.ds(g * 16, 16)] = xg * wf[d] + bf[d]
            return 0
        lax.fori_loop(0, GROUPS, num_body, 0)
        pltpu.async_copy(
            nbuf.at[sel], out_hbm.at[f, :, pl.ds(base, BPW)], nsem[sel])

    pltpu.make_async_copy(
        tab_hbm.at[:, pl.ds(sid, 1), pl.ds(0, CACHE)],
        sharedv.at[:, pl.ds(sid, 1), :], cachesem).wait()
    plsc.subcore_barrier()
    pltpu.sync_copy(sharedv, cachev)

    def compute_cat(cf, buf):
        cf16 = jnp.full((16,), cf, jnp.int32)

        def grp_body(g, _):
            idxg = gidx[cf, g >> 3, pl.ds((g & 7) * 16, 16)]
            idxc = jnp.minimum(idxg, CACHE - 1)
            for d in range(D):
                val = plsc.load_gather(
                    cachev, [cf16, jnp.full((16,), d, jnp.int32), idxc])
                buf[d, pl.ds(g * 16, 16)] = val
            return 0
        lax.fori_loop(0, GROUPS, grp_body, 0)

        for k in range(NK):
            @pl.when(mflag[cf, k] >= CACHE)
            def _():
                for d in range(D):
                    pltpu.async_copy(
                        tab_hbm.at[cf, d].at[gidx.at[cf, k]],
                        buf.at[d, pl.ds(k * KCH, KCH)],
                        gsem)
                pltpu.make_async_copy(
                    tab_hbm.at[0, :, pl.ds(0, KCH)],
                    buf.at[:, pl.ds(0, KCH)], gsem).wait()

    def cat_pair(j, _):
        cf0 = 2 * j
        cf1 = 2 * j + 1

        @pl.when(j >= 1)
        def _():
            pltpu.make_async_copy(
                cbuf.at[0], out_hbm.at[N_NUM, :, pl.ds(base, BPW)],
                csem[0]).wait()
        compute_cat(cf0, cbuf.at[0])
        pltpu.async_copy(
            cbuf.at[0], out_hbm.at[N_NUM + cf0, :, pl.ds(base, BPW)], csem[0])

        @pl.when(j >= 1)
        def _():
            pltpu.make_async_copy(
                cbuf.at[1], out_hbm.at[N_NUM, :, pl.ds(base, BPW)],
                csem[1]).wait()
        compute_cat(cf1, cbuf.at[1])
        pltpu.async_copy(
            cbuf.at[1], out_hbm.at[N_NUM + cf1, :, pl.ds(base, BPW)], csem[1])
        return 0
    lax.fori_loop(0, N_CAT // 2, cat_pair, 0)

    for sel in range(2):
        pltpu.make_async_copy(
            nbuf.at[sel], out_hbm.at[0, :, pl.ds(base, BPW)],
            nsem[sel]).wait()
        pltpu.make_async_copy(
            cbuf.at[sel], out_hbm.at[N_NUM, :, pl.ds(base, BPW)],
            csem[sel]).wait()


@functools.partial(
    pl.kernel,
    out_type=jax.ShapeDtypeStruct((N_FEAT, D, B), jnp.float32),
    mesh=plsc.VectorSubcoreMesh(core_axis_name="c", subcore_axis_name="s"),
    compiler_params=pltpu.CompilerParams(
        use_tc_tiling_on_sc=False, needs_layout_passes=False),
    scratch_types=[
        pltpu.VMEM((N_FEAT, BPW), jnp.float32),
        pltpu.VMEM((N_CAT, NK, KCH), jnp.int32),
        pltpu.VMEM((N_NUM, D), jnp.float32),
        pltpu.VMEM((N_NUM, D), jnp.float32),
        pltpu.VMEM_SHARED((N_CAT, D, CACHE), jnp.float32),
        pltpu.VMEM((N_CAT, D, CACHE), jnp.float32),
        pltpu.VMEM((2, D, BPW), jnp.float32),
        pltpu.VMEM((2, D, BPW), jnp.float32),
        pltpu.SMEM((N_CAT, NK), jnp.int32),
        pltpu.SemaphoreType.DMA,
        pltpu.SemaphoreType.DMA,
        pltpu.SemaphoreType.DMA,
        pltpu.SemaphoreType.DMA,
        pltpu.SemaphoreType.DMA,
        pltpu.SemaphoreType.DMA,
    ],
)
def _sc_embed(x_hbm, w_hbm, b_hbm, tab_hbm, out_hbm,
              xv, gidx, wv, bv, sharedv, cachev, cbuf, nbuf, mflag,
              gsem, cachesem, csem0, csem1, nsem0, nsem1):
    _body(x_hbm, w_hbm, b_hbm, tab_hbm, out_hbm,
          xv, gidx, wv, bv, sharedv, cachev, cbuf, nbuf, mflag,
          gsem, cachesem, csem0, csem1, nsem0, nsem1)


def kernel(x, num_W, num_b, cat_tables):
    out_t = _sc_embed(x.T, num_W, num_b, cat_tables.transpose(0, 2, 1))
    return out_t.transpose(2, 0, 1)
